# baseline (speedup 1.0000x reference)
import jax
import jax.numpy as jnp
from jax import lax
from jax.experimental import pallas as pl
from jax.experimental.pallas import tpu as pltpu
from jax.experimental.pallas import tpu_sc as plsc

N = 10000
DIN = 128
DH = 256
DOUT = 128
E0 = 320000

NC = 2
NS = 16
CHUNK = 128
ROWS_PER_TILE = 640
NPAD = NS * ROWS_PER_TILE
EPAD = 32768 * 10
BR = 1024


def _mesh():
    return plsc.VectorSubcoreMesh(core_axis_name="c", subcore_axis_name="s")


def _make_agg(split_edges: bool, idx_blk: int = 80):
    per_core = EPAD // 2 if split_edges else EPAD
    per_tile = per_core // NS
    n_chunks = per_tile // CHUNK
    n_outer = n_chunks // idx_blk
    estarts = (0, EPAD // 2) if split_edges else (0, 0)

    def body(t0, t1, src, dst, i0, i1, out, sidx, didx, bufa, acc, sema):
        cid = lax.axis_index("c")
        sid = lax.axis_index("s")

        def work(table, init, estart, core):
            cb = estart // CHUNK + sid * n_chunks
            r0 = sid * ROWS_PER_TILE
            pltpu.sync_copy(init.at[pl.ds(r0, ROWS_PER_TILE)],
                            acc.at[pl.ds(r0, ROWS_PER_TILE)])
            plsc.subcore_barrier()

            for h in range(n_outer):
                pltpu.sync_copy(src.at[pl.ds(cb + h * idx_blk, idx_blk)], sidx)
                pltpu.sync_copy(dst.at[pl.ds(cb + h * idx_blk, idx_blk)], didx)
                @pl.loop(0, idx_blk)
                def _(j):
                    pltpu.async_copy(table.at[sidx.at[j]], bufa, sema).wait()
                    pltpu.sync_copy(bufa, acc.at[didx.at[j]], add=True)

            plsc.subcore_barrier()
            pltpu.sync_copy(acc.at[pl.ds(r0, ROWS_PER_TILE)],
                            out.at[core, pl.ds(r0, ROWS_PER_TILE)])

        @pl.when(cid == 0)
        def _():
            work(t0, i0, estarts[0], 0)

        @pl.when(cid == 1)
        def _():
            work(t1, i1, estarts[1], 1)

    return pl.kernel(
        body,
        jax.ShapeDtypeStruct((2, NPAD, DIN), jnp.float32),
        mesh=_mesh(),
        scratch_types=[
            pltpu.VMEM((idx_blk, CHUNK), jnp.int32),
            pltpu.VMEM((idx_blk, CHUNK), jnp.int32),
            pltpu.VMEM((CHUNK, DIN), jnp.float32),
            pltpu.VMEM_SHARED((NPAD, DIN), jnp.float32),
            pltpu.SemaphoreType.DMA,
        ],
    )


def _make_deg():
    per_tile = (EPAD // 2) // NS
    n_chunks = per_tile // CHUNK

    def body(dst, ones, zinit, out, didx, onesbuf, acc, sem):
        cid = lax.axis_index("c")
        sid = lax.axis_index("s")

        def work(estart, core):
            cb = estart // CHUNK + sid * n_chunks
            pltpu.sync_copy(dst.at[pl.ds(cb, n_chunks)], didx)
            pltpu.sync_copy(ones, onesbuf)
            r0 = sid * ROWS_PER_TILE
            pltpu.sync_copy(zinit.at[pl.ds(r0, ROWS_PER_TILE)],
                            acc.at[pl.ds(r0, ROWS_PER_TILE)])
            plsc.subcore_barrier()

            @pl.loop(0, n_chunks)
            def _(j):
                pltpu.sync_copy(onesbuf, acc.at[didx.at[j]], add=True)

            plsc.subcore_barrier()
            pltpu.sync_copy(acc.at[pl.ds(r0, ROWS_PER_TILE)],
                            out.at[core, pl.ds(r0, ROWS_PER_TILE)])

        @pl.when(cid == 0)
        def _():
            work(0, 0)

        @pl.when(cid == 1)
        def _():
            work(EPAD // 2, 1)

    return pl.kernel(
        body,
        jax.ShapeDtypeStruct((2, NPAD, DIN), jnp.float32),
        mesh=_mesh(),
        scratch_types=[
            pltpu.VMEM((n_chunks, CHUNK), jnp.int32),
            pltpu.VMEM((CHUNK, DIN), jnp.float32),
            pltpu.VMEM_SHARED((NPAD, DIN), jnp.float32),
            pltpu.SemaphoreType.DMA,
        ],
    )


_agg_split = _make_agg(True)
_agg_full = _make_agg(False)
_deg = _make_deg()


def _blk(r, c):
    return pl.BlockSpec((r, c), lambda i: (i, 0))


def _full(r, c):
    return pl.BlockSpec((r, c), lambda i: (0, 0))


def _tc_a_body(x, w, b, dv, o):
    h = jax.nn.sigmoid(
        jnp.dot(x[...], w[...], preferred_element_type=jnp.float32) + b[...])
    o[...] = h * dv[...]


_tc_a = pl.pallas_call(
    _tc_a_body,
    out_shape=jax.ShapeDtypeStruct((NPAD, DIN), jnp.float32),
    grid=(NPAD // BR,),
    in_specs=[_blk(BR, DIN), _full(DIN, DIN), _full(1, DIN), _blk(BR, 1)],
    out_specs=_blk(BR, DIN),
)


def _tc_b_body(a0, a1, dv, w, b, o):
    t = (a0[...] + a1[...]) * dv[...]
    h = jnp.maximum(
        jnp.dot(t, w[...], preferred_element_type=jnp.float32) + b[...], 0.0)
    o[...] = h * dv[...]


_tc_b = pl.pallas_call(
    _tc_b_body,
    out_shape=jax.ShapeDtypeStruct((NPAD, DH), jnp.float32),
    grid=(NPAD // BR,),
    in_specs=[_blk(BR, DIN), _blk(BR, DIN), _blk(BR, 1),
              _full(DIN, DH), _full(1, DH)],
    out_specs=_blk(BR, DH),
)


def _tc_c_body(a0, a1, dv, w2a, w2b, b2, w3, o):
    t0 = a0[...] * dv[...]
    t1 = a1[...] * dv[...]
    h = jnp.maximum(
        jnp.dot(t0, w2a[...], preferred_element_type=jnp.float32)
        + jnp.dot(t1, w2b[...], preferred_element_type=jnp.float32)
        + b2[...], 0.0)
    o[...] = jnp.dot(h, w3[...], preferred_element_type=jnp.float32) * dv[...]


_tc_c = pl.pallas_call(
    _tc_c_body,
    out_shape=jax.ShapeDtypeStruct((NPAD, DOUT), jnp.float32),
    grid=(NPAD // BR,),
    in_specs=[_blk(BR, DIN), _blk(BR, DIN), _blk(BR, 1),
              _full(DIN, DH), _full(DIN, DH), _full(1, DH), _full(DH, DOUT)],
    out_specs=_blk(BR, DOUT),
)


def _tc_d_body(a0, a1, dv, b3, wo, bo, o):
    t = (a0[...] + a1[...]) * dv[...]
    h = jnp.maximum(t + b3[...], 0.0)
    o[...] = jnp.maximum(
        jnp.dot(h, wo[...], preferred_element_type=jnp.float32) + bo[...], 0.0)


_tc_d = pl.pallas_call(
    _tc_d_body,
    out_shape=jax.ShapeDtypeStruct((NPAD, DOUT), jnp.float32),
    grid=(NPAD // BR,),
    in_specs=[_blk(BR, DOUT), _blk(BR, DOUT), _blk(BR, 1),
              _full(1, DOUT), _full(DOUT, DOUT), _full(1, DOUT)],
    out_specs=_blk(BR, DOUT),
)


@jax.jit
def kernel(x, edge_index, W_in, b_in, W1, b1, W2, b2, W3, b3, W_out, b_out):
    f32 = jnp.float32
    src = edge_index[0]
    dst = edge_index[1]
    pad = jnp.full((EPAD - E0,), N, jnp.int32)
    src2d = jnp.concatenate([src, pad]).reshape(EPAD // CHUNK, CHUNK)
    dst2d = jnp.concatenate([dst, pad]).reshape(EPAD // CHUNK, CHUNK)
    xp = jnp.zeros((NPAD, DIN), f32).at[:N].set(x)
    ones128 = jnp.ones((CHUNK, DIN), f32)
    z128 = jnp.zeros((NPAD, DIN), f32)

    degp = _deg(dst2d, ones128, z128)
    deg = degp[0, :, 0] + degp[1, :, 0] + 1.0
    dv = lax.rsqrt(deg).reshape(NPAD, 1)

    g1 = _tc_a(xp, W_in, b_in.reshape(1, DIN), dv)
    a1 = _agg_split(g1, g1, src2d, dst2d, g1, z128)
    g2 = _tc_b(a1[0], a1[1], dv, W1, b1.reshape(1, DH))
    a2 = _agg_full(g2[:, :DIN], g2[:, DIN:], src2d, dst2d,
                   g2[:, :DIN], g2[:, DIN:])
    g3 = _tc_c(a2[0], a2[1], dv, W2[:DIN], W2[DIN:], b2.reshape(1, DH), W3)
    a3 = _agg_split(g3, g3, src2d, dst2d, g3, z128)
    y = _tc_d(a3[0], a3[1], dv, b3.reshape(1, DOUT), W_out,
              b_out.reshape(1, DOUT))
    return y[:N]


# speedup vs baseline: 1.0350x; 1.0350x over previous
import jax
import jax.numpy as jnp
from jax import lax
from jax.experimental import pallas as pl
from jax.experimental.pallas import tpu as pltpu
from jax.experimental.pallas import tpu_sc as plsc

N = 10000
DIN = 128
DH = 256
DOUT = 128
E0 = 320000

NC = 2
NS = 16
CHUNK = 128
ROWS_PER_TILE = 640
NPAD = NS * ROWS_PER_TILE
EPAD = 32768 * 10
BR = 1024


def _mesh():
    return plsc.VectorSubcoreMesh(core_axis_name="c", subcore_axis_name="s")


def _make_agg(split_edges: bool, idx_blk: int = 40):
    per_core = EPAD // 2 if split_edges else EPAD
    per_tile = per_core // NS
    n_chunks = per_tile // CHUNK
    n_outer = n_chunks // idx_blk
    estarts = (0, EPAD // 2) if split_edges else (0, 0)

    def body(t0, t1, src, dst, i0, i1, out, sidx, didx, bufa, bufb, acc,
             sema, semb):
        cid = lax.axis_index("c")
        sid = lax.axis_index("s")

        def work(table, init, estart, core):
            cb = estart // CHUNK + sid * n_chunks
            r0 = sid * ROWS_PER_TILE
            pltpu.sync_copy(init.at[pl.ds(r0, ROWS_PER_TILE)],
                            acc.at[pl.ds(r0, ROWS_PER_TILE)])
            plsc.subcore_barrier()

            for h in range(n_outer):
                pltpu.sync_copy(src.at[pl.ds(cb + h * idx_blk, idx_blk)], sidx)
                pltpu.sync_copy(dst.at[pl.ds(cb + h * idx_blk, idx_blk)], didx)
                @pl.loop(0, idx_blk, step=2)
                def _(j):
                    cpa = pltpu.async_copy(table.at[sidx.at[j]], bufa, sema)
                    cpb = pltpu.async_copy(table.at[sidx.at[j + 1]], bufb,
                                           semb)
                    cpa.wait()
                    pltpu.sync_copy(bufa, acc.at[didx.at[j]], add=True)
                    cpb.wait()
                    pltpu.sync_copy(bufb, acc.at[didx.at[j + 1]], add=True)

            plsc.subcore_barrier()
            pltpu.sync_copy(acc.at[pl.ds(r0, ROWS_PER_TILE)],
                            out.at[core, pl.ds(r0, ROWS_PER_TILE)])

        @pl.when(cid == 0)
        def _():
            work(t0, i0, estarts[0], 0)

        @pl.when(cid == 1)
        def _():
            work(t1, i1, estarts[1], 1)

    return pl.kernel(
        body,
        jax.ShapeDtypeStruct((2, NPAD, DIN), jnp.float32),
        mesh=_mesh(),
        scratch_types=[
            pltpu.VMEM((idx_blk, CHUNK), jnp.int32),
            pltpu.VMEM((idx_blk, CHUNK), jnp.int32),
            pltpu.VMEM((CHUNK, DIN), jnp.float32),
            pltpu.VMEM((CHUNK, DIN), jnp.float32),
            pltpu.VMEM_SHARED((NPAD, DIN), jnp.float32),
            pltpu.SemaphoreType.DMA,
            pltpu.SemaphoreType.DMA,
        ],
    )


def _make_deg():
    per_tile = (EPAD // 2) // NS
    n_chunks = per_tile // CHUNK

    def body(dst, ones, zinit, out, didx, onesbuf, acc, sem):
        cid = lax.axis_index("c")
        sid = lax.axis_index("s")

        def work(estart, core):
            cb = estart // CHUNK + sid * n_chunks
            pltpu.sync_copy(dst.at[pl.ds(cb, n_chunks)], didx)
            pltpu.sync_copy(ones, onesbuf)
            r0 = sid * ROWS_PER_TILE
            pltpu.sync_copy(zinit.at[pl.ds(r0, ROWS_PER_TILE)],
                            acc.at[pl.ds(r0, ROWS_PER_TILE)])
            plsc.subcore_barrier()

            @pl.loop(0, n_chunks)
            def _(j):
                pltpu.sync_copy(onesbuf, acc.at[didx.at[j]], add=True)

            plsc.subcore_barrier()
            pltpu.sync_copy(acc.at[pl.ds(r0, ROWS_PER_TILE)],
                            out.at[core, pl.ds(r0, ROWS_PER_TILE)])

        @pl.when(cid == 0)
        def _():
            work(0, 0)

        @pl.when(cid == 1)
        def _():
            work(EPAD // 2, 1)

    return pl.kernel(
        body,
        jax.ShapeDtypeStruct((2, NPAD, DIN), jnp.float32),
        mesh=_mesh(),
        scratch_types=[
            pltpu.VMEM((n_chunks, CHUNK), jnp.int32),
            pltpu.VMEM((CHUNK, DIN), jnp.float32),
            pltpu.VMEM_SHARED((NPAD, DIN), jnp.float32),
            pltpu.SemaphoreType.DMA,
        ],
    )


_agg_split = _make_agg(True)
_agg_full = _make_agg(False)
_deg = _make_deg()


def _blk(r, c):
    return pl.BlockSpec((r, c), lambda i: (i, 0))


def _full(r, c):
    return pl.BlockSpec((r, c), lambda i: (0, 0))


def _tc_a_body(x, w, b, dv, o):
    h = jax.nn.sigmoid(
        jnp.dot(x[...], w[...], preferred_element_type=jnp.float32) + b[...])
    o[...] = h * dv[...]


_tc_a = pl.pallas_call(
    _tc_a_body,
    out_shape=jax.ShapeDtypeStruct((NPAD, DIN), jnp.float32),
    grid=(NPAD // BR,),
    in_specs=[_blk(BR, DIN), _full(DIN, DIN), _full(1, DIN), _blk(BR, 1)],
    out_specs=_blk(BR, DIN),
)


def _tc_b_body(a0, a1, dv, w, b, o):
    t = (a0[...] + a1[...]) * dv[...]
    h = jnp.maximum(
        jnp.dot(t, w[...], preferred_element_type=jnp.float32) + b[...], 0.0)
    o[...] = h * dv[...]


_tc_b = pl.pallas_call(
    _tc_b_body,
    out_shape=jax.ShapeDtypeStruct((NPAD, DH), jnp.float32),
    grid=(NPAD // BR,),
    in_specs=[_blk(BR, DIN), _blk(BR, DIN), _blk(BR, 1),
              _full(DIN, DH), _full(1, DH)],
    out_specs=_blk(BR, DH),
)


def _tc_c_body(a0, a1, dv, w2a, w2b, b2, w3, o):
    t0 = a0[...] * dv[...]
    t1 = a1[...] * dv[...]
    h = jnp.maximum(
        jnp.dot(t0, w2a[...], preferred_element_type=jnp.float32)
        + jnp.dot(t1, w2b[...], preferred_element_type=jnp.float32)
        + b2[...], 0.0)
    o[...] = jnp.dot(h, w3[...], preferred_element_type=jnp.float32) * dv[...]


_tc_c = pl.pallas_call(
    _tc_c_body,
    out_shape=jax.ShapeDtypeStruct((NPAD, DOUT), jnp.float32),
    grid=(NPAD // BR,),
    in_specs=[_blk(BR, DIN), _blk(BR, DIN), _blk(BR, 1),
              _full(DIN, DH), _full(DIN, DH), _full(1, DH), _full(DH, DOUT)],
    out_specs=_blk(BR, DOUT),
)


def _tc_d_body(a0, a1, dv, b3, wo, bo, o):
    t = (a0[...] + a1[...]) * dv[...]
    h = jnp.maximum(t + b3[...], 0.0)
    o[...] = jnp.maximum(
        jnp.dot(h, wo[...], preferred_element_type=jnp.float32) + bo[...], 0.0)


_tc_d = pl.pallas_call(
    _tc_d_body,
    out_shape=jax.ShapeDtypeStruct((NPAD, DOUT), jnp.float32),
    grid=(NPAD // BR,),
    in_specs=[_blk(BR, DOUT), _blk(BR, DOUT), _blk(BR, 1),
              _full(1, DOUT), _full(DOUT, DOUT), _full(1, DOUT)],
    out_specs=_blk(BR, DOUT),
)


@jax.jit
def kernel(x, edge_index, W_in, b_in, W1, b1, W2, b2, W3, b3, W_out, b_out):
    f32 = jnp.float32
    src = edge_index[0]
    dst = edge_index[1]
    pad = jnp.full((EPAD - E0,), N, jnp.int32)
    src2d = jnp.concatenate([src, pad]).reshape(EPAD // CHUNK, CHUNK)
    dst2d = jnp.concatenate([dst, pad]).reshape(EPAD // CHUNK, CHUNK)
    xp = jnp.zeros((NPAD, DIN), f32).at[:N].set(x)
    ones128 = jnp.ones((CHUNK, DIN), f32)
    z128 = jnp.zeros((NPAD, DIN), f32)

    degp = _deg(dst2d, ones128, z128)
    deg = degp[0, :, 0] + degp[1, :, 0] + 1.0
    dv = lax.rsqrt(deg).reshape(NPAD, 1)

    g1 = _tc_a(xp, W_in, b_in.reshape(1, DIN), dv)
    a1 = _agg_split(g1, g1, src2d, dst2d, g1, z128)
    g2 = _tc_b(a1[0], a1[1], dv, W1, b1.reshape(1, DH))
    a2 = _agg_full(g2[:, :DIN], g2[:, DIN:], src2d, dst2d,
                   g2[:, :DIN], g2[:, DIN:])
    g3 = _tc_c(a2[0], a2[1], dv, W2[:DIN], W2[DIN:], b2.reshape(1, DH), W3)
    a3 = _agg_split(g3, g3, src2d, dst2d, g3, z128)
    y = _tc_d(a3[0], a3[1], dv, b3.reshape(1, DOUT), W_out,
              b_out.reshape(1, DOUT))
    return y[:N]


# speedup vs baseline: 1.0391x; 1.0039x over previous
import jax
import jax.numpy as jnp
from jax import lax
from jax.experimental import pallas as pl
from jax.experimental.pallas import tpu as pltpu
from jax.experimental.pallas import tpu_sc as plsc

N = 10000
DIN = 128
DH = 256
DOUT = 128
E0 = 320000

NC = 2
NS = 16
CHUNK = 128
ROWS_PER_TILE = 640
NPAD = NS * ROWS_PER_TILE
EPAD = 32768 * 10
BR = 1024


def _mesh():
    return plsc.VectorSubcoreMesh(core_axis_name="c", subcore_axis_name="s")


def _make_agg(split_edges: bool, idx_blk: int = 40):
    per_core = EPAD // 2 if split_edges else EPAD
    per_tile = per_core // NS
    n_chunks = per_tile // CHUNK
    n_outer = n_chunks // idx_blk
    estarts = (0, EPAD // 2) if split_edges else (0, 0)

    def body(t0, t1, src, dst, i0, i1, out, sidx, didx, bufa, bufb, acc,
             sema, semb):
        cid = lax.axis_index("c")
        sid = lax.axis_index("s")

        def work(table, init, estart, core):
            cb = estart // CHUNK + sid * n_chunks
            r0 = sid * ROWS_PER_TILE
            pltpu.sync_copy(init.at[pl.ds(r0, ROWS_PER_TILE)],
                            acc.at[pl.ds(r0, ROWS_PER_TILE)])
            plsc.subcore_barrier()

            for h in range(n_outer):
                pltpu.sync_copy(src.at[pl.ds(cb + h * idx_blk, idx_blk)], sidx)
                pltpu.sync_copy(dst.at[pl.ds(cb + h * idx_blk, idx_blk)], didx)
                @pl.loop(0, idx_blk, step=2)
                def _(j):
                    cpa = pltpu.async_copy(table.at[sidx.at[j]], bufa, sema)
                    cpb = pltpu.async_copy(table.at[sidx.at[j + 1]], bufb,
                                           semb)
                    cpa.wait()
                    spa = pltpu.async_copy(bufa, acc.at[didx.at[j]], sema,
                                           add=True)
                    cpb.wait()
                    spb = pltpu.async_copy(bufb, acc.at[didx.at[j + 1]], semb,
                                           add=True)
                    spa.wait()
                    spb.wait()

            plsc.subcore_barrier()
            pltpu.sync_copy(acc.at[pl.ds(r0, ROWS_PER_TILE)],
                            out.at[core, pl.ds(r0, ROWS_PER_TILE)])

        @pl.when(cid == 0)
        def _():
            work(t0, i0, estarts[0], 0)

        @pl.when(cid == 1)
        def _():
            work(t1, i1, estarts[1], 1)

    return pl.kernel(
        body,
        jax.ShapeDtypeStruct((2, NPAD, DIN), jnp.float32),
        mesh=_mesh(),
        scratch_types=[
            pltpu.VMEM((idx_blk, CHUNK), jnp.int32),
            pltpu.VMEM((idx_blk, CHUNK), jnp.int32),
            pltpu.VMEM((CHUNK, DIN), jnp.float32),
            pltpu.VMEM((CHUNK, DIN), jnp.float32),
            pltpu.VMEM_SHARED((NPAD, DIN), jnp.float32),
            pltpu.SemaphoreType.DMA,
            pltpu.SemaphoreType.DMA,
        ],
    )


def _make_deg():
    per_tile = (EPAD // 2) // NS
    n_chunks = per_tile // CHUNK

    def body(dst, ones, zinit, out, didx, onesbuf, acc, sem):
        cid = lax.axis_index("c")
        sid = lax.axis_index("s")

        def work(estart, core):
            cb = estart // CHUNK + sid * n_chunks
            pltpu.sync_copy(dst.at[pl.ds(cb, n_chunks)], didx)
            pltpu.sync_copy(ones, onesbuf)
            r0 = sid * ROWS_PER_TILE
            pltpu.sync_copy(zinit.at[pl.ds(r0, ROWS_PER_TILE)],
                            acc.at[pl.ds(r0, ROWS_PER_TILE)])
            plsc.subcore_barrier()

            @pl.loop(0, n_chunks)
            def _(j):
                pltpu.sync_copy(onesbuf, acc.at[didx.at[j]], add=True)

            plsc.subcore_barrier()
            pltpu.sync_copy(acc.at[pl.ds(r0, ROWS_PER_TILE)],
                            out.at[core, pl.ds(r0, ROWS_PER_TILE)])

        @pl.when(cid == 0)
        def _():
            work(0, 0)

        @pl.when(cid == 1)
        def _():
            work(EPAD // 2, 1)

    return pl.kernel(
        body,
        jax.ShapeDtypeStruct((2, NPAD, DIN), jnp.float32),
        mesh=_mesh(),
        scratch_types=[
            pltpu.VMEM((n_chunks, CHUNK), jnp.int32),
            pltpu.VMEM((CHUNK, DIN), jnp.float32),
            pltpu.VMEM_SHARED((NPAD, DIN), jnp.float32),
            pltpu.SemaphoreType.DMA,
        ],
    )


_agg_split = _make_agg(True)
_agg_full = _make_agg(False)
_deg = _make_deg()


def _blk(r, c):
    return pl.BlockSpec((r, c), lambda i: (i, 0))


def _full(r, c):
    return pl.BlockSpec((r, c), lambda i: (0, 0))


def _tc_a_body(x, w, b, dv, o):
    h = jax.nn.sigmoid(
        jnp.dot(x[...], w[...], preferred_element_type=jnp.float32) + b[...])
    o[...] = h * dv[...]


_tc_a = pl.pallas_call(
    _tc_a_body,
    out_shape=jax.ShapeDtypeStruct((NPAD, DIN), jnp.float32),
    grid=(NPAD // BR,),
    in_specs=[_blk(BR, DIN), _full(DIN, DIN), _full(1, DIN), _blk(BR, 1)],
    out_specs=_blk(BR, DIN),
)


def _tc_b_body(a0, a1, dv, w, b, o):
    t = (a0[...] + a1[...]) * dv[...]
    h = jnp.maximum(
        jnp.dot(t, w[...], preferred_element_type=jnp.float32) + b[...], 0.0)
    o[...] = h * dv[...]


_tc_b = pl.pallas_call(
    _tc_b_body,
    out_shape=jax.ShapeDtypeStruct((NPAD, DH), jnp.float32),
    grid=(NPAD // BR,),
    in_specs=[_blk(BR, DIN), _blk(BR, DIN), _blk(BR, 1),
              _full(DIN, DH), _full(1, DH)],
    out_specs=_blk(BR, DH),
)


def _tc_c_body(a0, a1, dv, w2a, w2b, b2, w3, o):
    t0 = a0[...] * dv[...]
    t1 = a1[...] * dv[...]
    h = jnp.maximum(
        jnp.dot(t0, w2a[...], preferred_element_type=jnp.float32)
        + jnp.dot(t1, w2b[...], preferred_element_type=jnp.float32)
        + b2[...], 0.0)
    o[...] = jnp.dot(h, w3[...], preferred_element_type=jnp.float32) * dv[...]


_tc_c = pl.pallas_call(
    _tc_c_body,
    out_shape=jax.ShapeDtypeStruct((NPAD, DOUT), jnp.float32),
    grid=(NPAD // BR,),
    in_specs=[_blk(BR, DIN), _blk(BR, DIN), _blk(BR, 1),
              _full(DIN, DH), _full(DIN, DH), _full(1, DH), _full(DH, DOUT)],
    out_specs=_blk(BR, DOUT),
)


def _tc_d_body(a0, a1, dv, b3, wo, bo, o):
    t = (a0[...] + a1[...]) * dv[...]
    h = jnp.maximum(t + b3[...], 0.0)
    o[...] = jnp.maximum(
        jnp.dot(h, wo[...], preferred_element_type=jnp.float32) + bo[...], 0.0)


_tc_d = pl.pallas_call(
    _tc_d_body,
    out_shape=jax.ShapeDtypeStruct((NPAD, DOUT), jnp.float32),
    grid=(NPAD // BR,),
    in_specs=[_blk(BR, DOUT), _blk(BR, DOUT), _blk(BR, 1),
              _full(1, DOUT), _full(DOUT, DOUT), _full(1, DOUT)],
    out_specs=_blk(BR, DOUT),
)


@jax.jit
def kernel(x, edge_index, W_in, b_in, W1, b1, W2, b2, W3, b3, W_out, b_out):
    f32 = jnp.float32
    src = edge_index[0]
    dst = edge_index[1]
    pad = jnp.full((EPAD - E0,), N, jnp.int32)
    src2d = jnp.concatenate([src, pad]).reshape(EPAD // CHUNK, CHUNK)
    dst2d = jnp.concatenate([dst, pad]).reshape(EPAD // CHUNK, CHUNK)
    xp = jnp.zeros((NPAD, DIN), f32).at[:N].set(x)
    ones128 = jnp.ones((CHUNK, DIN), f32)
    z128 = jnp.zeros((NPAD, DIN), f32)

    degp = _deg(dst2d, ones128, z128)
    deg = degp[0, :, 0] + degp[1, :, 0] + 1.0
    dv = lax.rsqrt(deg).reshape(NPAD, 1)

    g1 = _tc_a(xp, W_in, b_in.reshape(1, DIN), dv)
    a1 = _agg_split(g1, g1, src2d, dst2d, g1, z128)
    g2 = _tc_b(a1[0], a1[1], dv, W1, b1.reshape(1, DH))
    a2 = _agg_full(g2[:, :DIN], g2[:, DIN:], src2d, dst2d,
                   g2[:, :DIN], g2[:, DIN:])
    g3 = _tc_c(a2[0], a2[1], dv, W2[:DIN], W2[DIN:], b2.reshape(1, DH), W3)
    a3 = _agg_split(g3, g3, src2d, dst2d, g3, z128)
    y = _tc_d(a3[0], a3[1], dv, b3.reshape(1, DOUT), W_out,
              b_out.reshape(1, DOUT))
    return y[:N]


# speedup vs baseline: 2.7145x; 2.6123x over previous
import jax
import jax.numpy as jnp
from jax import lax
from jax.experimental import pallas as pl
from jax.experimental.pallas import tpu as pltpu
from jax.experimental.pallas import tpu_sc as plsc

N = 10000
DIN = 128
DH = 256
DOUT = 128
E0 = 320000

NC = 2
NS = 16
CHUNK = 128
ROWS_PER_TILE = 640
NPAD = NS * ROWS_PER_TILE
EPAD = 32768 * 10
BR = 1024


def _mesh():
    return plsc.VectorSubcoreMesh(core_axis_name="c", subcore_axis_name="s")


def _make_agg(split_edges: bool, idx_blk: int = 40):
    per_core = EPAD // 2 if split_edges else EPAD
    per_tile = per_core // NS
    n_chunks = per_tile // CHUNK
    n_outer = n_chunks // idx_blk
    estarts = (0, EPAD // 2) if split_edges else (0, 0)

    def body(t0, t1, src, dst, i0, i1, out, sidx, didx, bufa, bufb, acc,
             sema, semb):
        cid = lax.axis_index("c")
        sid = lax.axis_index("s")

        def work(table, init, estart, core):
            cb = estart // CHUNK + sid * n_chunks
            r0 = sid * ROWS_PER_TILE
            pltpu.sync_copy(init.at[pl.ds(r0, ROWS_PER_TILE)],
                            acc.at[pl.ds(r0, ROWS_PER_TILE)])
            plsc.subcore_barrier()

            for h in range(n_outer):
                pltpu.sync_copy(src.at[pl.ds(cb + h * idx_blk, idx_blk)], sidx)
                pltpu.sync_copy(dst.at[pl.ds(cb + h * idx_blk, idx_blk)], didx)
                @pl.loop(0, idx_blk, step=2)
                def _(j):
                    cpa = pltpu.async_copy(table.at[sidx.at[j]], bufa, sema)
                    cpb = pltpu.async_copy(table.at[sidx.at[j + 1]], bufb,
                                           semb)
                    cpa.wait()
                    spa = pltpu.async_copy(bufa, acc.at[didx.at[j]], sema,
                                           add=True)
                    cpb.wait()
                    spb = pltpu.async_copy(bufb, acc.at[didx.at[j + 1]], semb,
                                           add=True)
                    spa.wait()
                    spb.wait()

            plsc.subcore_barrier()
            pltpu.sync_copy(acc.at[pl.ds(r0, ROWS_PER_TILE)],
                            out.at[core, pl.ds(r0, ROWS_PER_TILE)])

        @pl.when(cid == 0)
        def _():
            work(t0, i0, estarts[0], 0)

        @pl.when(cid == 1)
        def _():
            work(t1, i1, estarts[1], 1)

    return pl.kernel(
        body,
        jax.ShapeDtypeStruct((2, NPAD, DIN), jnp.float32),
        mesh=_mesh(),
        scratch_types=[
            pltpu.VMEM((idx_blk, CHUNK), jnp.int32),
            pltpu.VMEM((idx_blk, CHUNK), jnp.int32),
            pltpu.VMEM((CHUNK, DIN), jnp.float32),
            pltpu.VMEM((CHUNK, DIN), jnp.float32),
            pltpu.VMEM_SHARED((NPAD, DIN), jnp.float32),
            pltpu.SemaphoreType.DMA,
            pltpu.SemaphoreType.DMA,
        ],
    )


def _make_deg():
    per_tile = (EPAD // 2) // NS
    n_chunks = per_tile // CHUNK

    def body(dst, ones, zinit, out, didx, onesbuf, acc, sem):
        cid = lax.axis_index("c")
        sid = lax.axis_index("s")

        def work(estart, core):
            cb = estart // CHUNK + sid * n_chunks
            pltpu.sync_copy(dst.at[pl.ds(cb, n_chunks)], didx)
            pltpu.sync_copy(ones, onesbuf)
            r0 = sid * ROWS_PER_TILE
            pltpu.sync_copy(zinit.at[pl.ds(r0, ROWS_PER_TILE)],
                            acc.at[pl.ds(r0, ROWS_PER_TILE)])
            plsc.subcore_barrier()

            @pl.loop(0, n_chunks)
            def _(j):
                pltpu.sync_copy(onesbuf, acc.at[didx.at[j]], add=True)

            plsc.subcore_barrier()
            pltpu.sync_copy(acc.at[pl.ds(r0, ROWS_PER_TILE)],
                            out.at[core, pl.ds(r0, ROWS_PER_TILE)])

        @pl.when(cid == 0)
        def _():
            work(0, 0)

        @pl.when(cid == 1)
        def _():
            work(EPAD // 2, 1)

    return pl.kernel(
        body,
        jax.ShapeDtypeStruct((2, NPAD, DIN), jnp.float32),
        mesh=_mesh(),
        scratch_types=[
            pltpu.VMEM((n_chunks, CHUNK), jnp.int32),
            pltpu.VMEM((CHUNK, DIN), jnp.float32),
            pltpu.VMEM_SHARED((NPAD, DIN), jnp.float32),
            pltpu.SemaphoreType.DMA,
        ],
    )


_agg_split = _make_agg(True)
_agg_full = _make_agg(False)
_deg = _make_deg()


def _blk(r, c):
    return pl.BlockSpec((r, c), lambda i: (i, 0))


def _full(r, c):
    return pl.BlockSpec((r, c), lambda i: (0, 0))


def _tc_a_body(x, w, b, dv, o):
    h = jax.nn.sigmoid(
        jnp.dot(x[...], w[...], preferred_element_type=jnp.float32) + b[...])
    o[...] = h * dv[...]


_tc_a = pl.pallas_call(
    _tc_a_body,
    out_shape=jax.ShapeDtypeStruct((NPAD, DIN), jnp.float32),
    grid=(NPAD // BR,),
    in_specs=[_blk(BR, DIN), _full(DIN, DIN), _full(1, DIN), _blk(BR, 1)],
    out_specs=_blk(BR, DIN),
)


def _tc_b_body(a0, a1, dv, w, b, o):
    t = (a0[...] + a1[...]) * dv[...]
    h = jnp.maximum(
        jnp.dot(t, w[...], preferred_element_type=jnp.float32) + b[...], 0.0)
    o[...] = h * dv[...]


_tc_b = pl.pallas_call(
    _tc_b_body,
    out_shape=jax.ShapeDtypeStruct((NPAD, DH), jnp.float32),
    grid=(NPAD // BR,),
    in_specs=[_blk(BR, DIN), _blk(BR, DIN), _blk(BR, 1),
              _full(DIN, DH), _full(1, DH)],
    out_specs=_blk(BR, DH),
)


def _tc_c_body(a0, a1, dv, w2a, w2b, b2, w3, o):
    t0 = a0[...] * dv[...]
    t1 = a1[...] * dv[...]
    h = jnp.maximum(
        jnp.dot(t0, w2a[...], preferred_element_type=jnp.float32)
        + jnp.dot(t1, w2b[...], preferred_element_type=jnp.float32)
        + b2[...], 0.0)
    o[...] = jnp.dot(h, w3[...], preferred_element_type=jnp.float32) * dv[...]


_tc_c = pl.pallas_call(
    _tc_c_body,
    out_shape=jax.ShapeDtypeStruct((NPAD, DOUT), jnp.float32),
    grid=(NPAD // BR,),
    in_specs=[_blk(BR, DIN), _blk(BR, DIN), _blk(BR, 1),
              _full(DIN, DH), _full(DIN, DH), _full(1, DH), _full(DH, DOUT)],
    out_specs=_blk(BR, DOUT),
)


def _tc_d_body(a0, a1, dv, b3, wo, bo, o):
    t = (a0[...] + a1[...]) * dv[...]
    h = jnp.maximum(t + b3[...], 0.0)
    o[...] = jnp.maximum(
        jnp.dot(h, wo[...], preferred_element_type=jnp.float32) + bo[...], 0.0)


_tc_d = pl.pallas_call(
    _tc_d_body,
    out_shape=jax.ShapeDtypeStruct((NPAD, DOUT), jnp.float32),
    grid=(NPAD // BR,),
    in_specs=[_blk(BR, DOUT), _blk(BR, DOUT), _blk(BR, 1),
              _full(1, DOUT), _full(DOUT, DOUT), _full(1, DOUT)],
    out_specs=_blk(BR, DOUT),
)


@jax.jit
def kernel(x, edge_index, W_in, b_in, W1, b1, W2, b2, W3, b3, W_out, b_out):
    f32 = jnp.float32
    src = edge_index[0]
    dst = edge_index[1]
    pad = N + (jnp.arange(EPAD - E0, dtype=jnp.int32) % (NPAD - N))
    src2d = jnp.concatenate([src, pad]).reshape(EPAD // CHUNK, CHUNK)
    dst2d = jnp.concatenate([dst, pad]).reshape(EPAD // CHUNK, CHUNK)
    xp = jnp.zeros((NPAD, DIN), f32).at[:N].set(x)
    ones128 = jnp.ones((CHUNK, DIN), f32)
    z128 = jnp.zeros((NPAD, DIN), f32)

    degp = _deg(dst2d, ones128, z128)
    deg = degp[0, :, 0] + degp[1, :, 0] + 1.0
    dv = lax.rsqrt(deg).reshape(NPAD, 1)

    g1 = _tc_a(xp, W_in, b_in.reshape(1, DIN), dv)
    a1 = _agg_split(g1, g1, src2d, dst2d, g1, z128)
    g2 = _tc_b(a1[0], a1[1], dv, W1, b1.reshape(1, DH))
    a2 = _agg_full(g2[:, :DIN], g2[:, DIN:], src2d, dst2d,
                   g2[:, :DIN], g2[:, DIN:])
    g3 = _tc_c(a2[0], a2[1], dv, W2[:DIN], W2[DIN:], b2.reshape(1, DH), W3)
    a3 = _agg_split(g3, g3, src2d, dst2d, g3, z128)
    y = _tc_d(a3[0], a3[1], dv, b3.reshape(1, DOUT), W_out,
              b_out.reshape(1, DOUT))
    return y[:N]


# speedup vs baseline: 2.9945x; 1.1032x over previous
import jax
import jax.numpy as jnp
from jax import lax
from jax.experimental import pallas as pl
from jax.experimental.pallas import tpu as pltpu
from jax.experimental.pallas import tpu_sc as plsc

N = 10000
DIN = 128
DH = 256
DOUT = 128
E0 = 320000

NC = 2
NS = 16
CHUNK = 128
ROWS_PER_TILE = 640
NPAD = NS * ROWS_PER_TILE
EPAD = 32768 * 10
BR = 1024


def _mesh():
    return plsc.VectorSubcoreMesh(core_axis_name="c", subcore_axis_name="s")


def _make_agg(split_edges: bool, idx_blk: int = 40):
    per_core = EPAD // 2 if split_edges else EPAD
    per_tile = per_core // NS
    n_chunks = per_tile // CHUNK
    n_outer = n_chunks // idx_blk
    estarts = (0, EPAD // 2) if split_edges else (0, 0)

    def body(t0, t1, src, dst, i0, i1, out, sidx, didx, bufa, bufb, acc,
             sema, semb, ssa, ssb):
        cid = lax.axis_index("c")
        sid = lax.axis_index("s")

        def work(table, init, estart, core):
            cb = estart // CHUNK + sid * n_chunks
            r0 = sid * ROWS_PER_TILE
            pltpu.sync_copy(init.at[pl.ds(r0, ROWS_PER_TILE)],
                            acc.at[pl.ds(r0, ROWS_PER_TILE)])
            plsc.subcore_barrier()

            for h in range(n_outer):
                pltpu.sync_copy(src.at[pl.ds(cb + h * idx_blk, idx_blk)], sidx)
                pltpu.sync_copy(dst.at[pl.ds(cb + h * idx_blk, idx_blk)], didx)
                pltpu.async_copy(table.at[sidx.at[0]], bufa, sema)

                @pl.loop(0, idx_blk, step=2)
                def _(j):
                    pltpu.make_async_copy(
                        table.at[sidx.at[j]], bufa, sema).wait()
                    pltpu.async_copy(bufa, acc.at[didx.at[j]], ssa, add=True)
                    pltpu.async_copy(table.at[sidx.at[j + 1]], bufb, semb)
                    pltpu.make_async_copy(
                        table.at[sidx.at[j + 1]], bufb, semb).wait()
                    pltpu.async_copy(bufb, acc.at[didx.at[j + 1]], ssb,
                                     add=True)
                    pltpu.make_async_copy(
                        bufa, acc.at[didx.at[j]], ssa).wait()

                    @pl.when(j + 2 < idx_blk)
                    def _():
                        pltpu.async_copy(table.at[sidx.at[j + 2]], bufa, sema)

                    pltpu.make_async_copy(
                        bufb, acc.at[didx.at[j + 1]], ssb).wait()

            plsc.subcore_barrier()
            pltpu.sync_copy(acc.at[pl.ds(r0, ROWS_PER_TILE)],
                            out.at[core, pl.ds(r0, ROWS_PER_TILE)])

        @pl.when(cid == 0)
        def _():
            work(t0, i0, estarts[0], 0)

        @pl.when(cid == 1)
        def _():
            work(t1, i1, estarts[1], 1)

    return pl.kernel(
        body,
        jax.ShapeDtypeStruct((2, NPAD, DIN), jnp.float32),
        mesh=_mesh(),
        scratch_types=[
            pltpu.VMEM((idx_blk, CHUNK), jnp.int32),
            pltpu.VMEM((idx_blk, CHUNK), jnp.int32),
            pltpu.VMEM((CHUNK, DIN), jnp.float32),
            pltpu.VMEM((CHUNK, DIN), jnp.float32),
            pltpu.VMEM_SHARED((NPAD, DIN), jnp.float32),
            pltpu.SemaphoreType.DMA,
            pltpu.SemaphoreType.DMA,
            pltpu.SemaphoreType.DMA,
            pltpu.SemaphoreType.DMA,
        ],
    )


def _make_deg():
    per_tile = (EPAD // 2) // NS
    n_chunks = per_tile // CHUNK

    def body(dst, ones, zinit, out, didx, onesbuf, acc, sem):
        cid = lax.axis_index("c")
        sid = lax.axis_index("s")

        def work(estart, core):
            cb = estart // CHUNK + sid * n_chunks
            pltpu.sync_copy(dst.at[pl.ds(cb, n_chunks)], didx)
            pltpu.sync_copy(ones, onesbuf)
            r0 = sid * ROWS_PER_TILE
            pltpu.sync_copy(zinit.at[pl.ds(r0, ROWS_PER_TILE)],
                            acc.at[pl.ds(r0, ROWS_PER_TILE)])
            plsc.subcore_barrier()

            @pl.loop(0, n_chunks)
            def _(j):
                pltpu.sync_copy(onesbuf, acc.at[didx.at[j]], add=True)

            plsc.subcore_barrier()
            pltpu.sync_copy(acc.at[pl.ds(r0, ROWS_PER_TILE)],
                            out.at[core, pl.ds(r0, ROWS_PER_TILE)])

        @pl.when(cid == 0)
        def _():
            work(0, 0)

        @pl.when(cid == 1)
        def _():
            work(EPAD // 2, 1)

    return pl.kernel(
        body,
        jax.ShapeDtypeStruct((2, NPAD, DIN), jnp.float32),
        mesh=_mesh(),
        scratch_types=[
            pltpu.VMEM((n_chunks, CHUNK), jnp.int32),
            pltpu.VMEM((CHUNK, DIN), jnp.float32),
            pltpu.VMEM_SHARED((NPAD, DIN), jnp.float32),
            pltpu.SemaphoreType.DMA,
        ],
    )


_agg_split = _make_agg(True)
_agg_full = _make_agg(False)
_deg = _make_deg()


def _blk(r, c):
    return pl.BlockSpec((r, c), lambda i: (i, 0))


def _full(r, c):
    return pl.BlockSpec((r, c), lambda i: (0, 0))


def _tc_a_body(x, w, b, dv, o):
    h = jax.nn.sigmoid(
        jnp.dot(x[...], w[...], preferred_element_type=jnp.float32) + b[...])
    o[...] = h * dv[...]


_tc_a = pl.pallas_call(
    _tc_a_body,
    out_shape=jax.ShapeDtypeStruct((NPAD, DIN), jnp.float32),
    grid=(NPAD // BR,),
    in_specs=[_blk(BR, DIN), _full(DIN, DIN), _full(1, DIN), _blk(BR, 1)],
    out_specs=_blk(BR, DIN),
)


def _tc_b_body(a0, a1, dv, w, b, o):
    t = (a0[...] + a1[...]) * dv[...]
    h = jnp.maximum(
        jnp.dot(t, w[...], preferred_element_type=jnp.float32) + b[...], 0.0)
    o[...] = h * dv[...]


_tc_b = pl.pallas_call(
    _tc_b_body,
    out_shape=jax.ShapeDtypeStruct((NPAD, DH), jnp.float32),
    grid=(NPAD // BR,),
    in_specs=[_blk(BR, DIN), _blk(BR, DIN), _blk(BR, 1),
              _full(DIN, DH), _full(1, DH)],
    out_specs=_blk(BR, DH),
)


def _tc_c_body(a0, a1, dv, w2a, w2b, b2, w3, o):
    t0 = a0[...] * dv[...]
    t1 = a1[...] * dv[...]
    h = jnp.maximum(
        jnp.dot(t0, w2a[...], preferred_element_type=jnp.float32)
        + jnp.dot(t1, w2b[...], preferred_element_type=jnp.float32)
        + b2[...], 0.0)
    o[...] = jnp.dot(h, w3[...], preferred_element_type=jnp.float32) * dv[...]


_tc_c = pl.pallas_call(
    _tc_c_body,
    out_shape=jax.ShapeDtypeStruct((NPAD, DOUT), jnp.float32),
    grid=(NPAD // BR,),
    in_specs=[_blk(BR, DIN), _blk(BR, DIN), _blk(BR, 1),
              _full(DIN, DH), _full(DIN, DH), _full(1, DH), _full(DH, DOUT)],
    out_specs=_blk(BR, DOUT),
)


def _tc_d_body(a0, a1, dv, b3, wo, bo, o):
    t = (a0[...] + a1[...]) * dv[...]
    h = jnp.maximum(t + b3[...], 0.0)
    o[...] = jnp.maximum(
        jnp.dot(h, wo[...], preferred_element_type=jnp.float32) + bo[...], 0.0)


_tc_d = pl.pallas_call(
    _tc_d_body,
    out_shape=jax.ShapeDtypeStruct((NPAD, DOUT), jnp.float32),
    grid=(NPAD // BR,),
    in_specs=[_blk(BR, DOUT), _blk(BR, DOUT), _blk(BR, 1),
              _full(1, DOUT), _full(DOUT, DOUT), _full(1, DOUT)],
    out_specs=_blk(BR, DOUT),
)


@jax.jit
def kernel(x, edge_index, W_in, b_in, W1, b1, W2, b2, W3, b3, W_out, b_out):
    f32 = jnp.float32
    src = edge_index[0]
    dst = edge_index[1]
    pad = N + (jnp.arange(EPAD - E0, dtype=jnp.int32) % (NPAD - N))
    src2d = jnp.concatenate([src, pad]).reshape(EPAD // CHUNK, CHUNK)
    dst2d = jnp.concatenate([dst, pad]).reshape(EPAD // CHUNK, CHUNK)
    xp = jnp.zeros((NPAD, DIN), f32).at[:N].set(x)
    ones128 = jnp.ones((CHUNK, DIN), f32)
    z128 = jnp.zeros((NPAD, DIN), f32)

    degp = _deg(dst2d, ones128, z128)
    deg = degp[0, :, 0] + degp[1, :, 0] + 1.0
    dv = lax.rsqrt(deg).reshape(NPAD, 1)

    g1 = _tc_a(xp, W_in, b_in.reshape(1, DIN), dv)
    a1 = _agg_split(g1, g1, src2d, dst2d, g1, z128)
    g2 = _tc_b(a1[0], a1[1], dv, W1, b1.reshape(1, DH))
    a2 = _agg_full(g2[:, :DIN], g2[:, DIN:], src2d, dst2d,
                   g2[:, :DIN], g2[:, DIN:])
    g3 = _tc_c(a2[0], a2[1], dv, W2[:DIN], W2[DIN:], b2.reshape(1, DH), W3)
    a3 = _agg_split(g3, g3, src2d, dst2d, g3, z128)
    y = _tc_d(a3[0], a3[1], dv, b3.reshape(1, DOUT), W_out,
              b_out.reshape(1, DOUT))
    return y[:N]


# speedup vs baseline: 3.0220x; 1.0092x over previous
import jax
import jax.numpy as jnp
from jax import lax
from jax.experimental import pallas as pl
from jax.experimental.pallas import tpu as pltpu
from jax.experimental.pallas import tpu_sc as plsc

N = 10000
DIN = 128
DH = 256
DOUT = 128
E0 = 320000

NC = 2
NS = 16
CHUNK = 128
ROWS_PER_TILE = 640
NPAD = NS * ROWS_PER_TILE
EPAD = 32768 * 10
BR = 2048


def _mesh():
    return plsc.VectorSubcoreMesh(core_axis_name="c", subcore_axis_name="s")


def _make_agg(split_edges: bool, idx_blk: int = 40):
    per_core = EPAD // 2 if split_edges else EPAD
    per_tile = per_core // NS
    n_chunks = per_tile // CHUNK
    n_outer = n_chunks // idx_blk
    estarts = (0, EPAD // 2) if split_edges else (0, 0)

    def body(t0, t1, src, dst, i0, i1, out, sidx, didx, bufa, bufb, acc,
             sema, semb, ssa, ssb):
        cid = lax.axis_index("c")
        sid = lax.axis_index("s")

        def work(table, init, estart, core):
            cb = estart // CHUNK + sid * n_chunks
            r0 = sid * ROWS_PER_TILE
            pltpu.sync_copy(init.at[pl.ds(r0, ROWS_PER_TILE)],
                            acc.at[pl.ds(r0, ROWS_PER_TILE)])
            plsc.subcore_barrier()

            for h in range(n_outer):
                pltpu.sync_copy(src.at[pl.ds(cb + h * idx_blk, idx_blk)], sidx)
                pltpu.sync_copy(dst.at[pl.ds(cb + h * idx_blk, idx_blk)], didx)
                pltpu.async_copy(table.at[sidx.at[0]], bufa, sema)

                @pl.loop(0, idx_blk, step=2)
                def _(j):
                    pltpu.make_async_copy(
                        table.at[sidx.at[j]], bufa, sema).wait()
                    pltpu.async_copy(bufa, acc.at[didx.at[j]], ssa, add=True)
                    pltpu.async_copy(table.at[sidx.at[j + 1]], bufb, semb)
                    pltpu.make_async_copy(
                        table.at[sidx.at[j + 1]], bufb, semb).wait()
                    pltpu.async_copy(bufb, acc.at[didx.at[j + 1]], ssb,
                                     add=True)
                    pltpu.make_async_copy(
                        bufa, acc.at[didx.at[j]], ssa).wait()

                    @pl.when(j + 2 < idx_blk)
                    def _():
                        pltpu.async_copy(table.at[sidx.at[j + 2]], bufa, sema)

                    pltpu.make_async_copy(
                        bufb, acc.at[didx.at[j + 1]], ssb).wait()

            plsc.subcore_barrier()
            pltpu.sync_copy(acc.at[pl.ds(r0, ROWS_PER_TILE)],
                            out.at[core, pl.ds(r0, ROWS_PER_TILE)])

        @pl.when(cid == 0)
        def _():
            work(t0, i0, estarts[0], 0)

        @pl.when(cid == 1)
        def _():
            work(t1, i1, estarts[1], 1)

    return pl.kernel(
        body,
        jax.ShapeDtypeStruct((2, NPAD, DIN), jnp.float32),
        mesh=_mesh(),
        scratch_types=[
            pltpu.VMEM((idx_blk, CHUNK), jnp.int32),
            pltpu.VMEM((idx_blk, CHUNK), jnp.int32),
            pltpu.VMEM((CHUNK, DIN), jnp.float32),
            pltpu.VMEM((CHUNK, DIN), jnp.float32),
            pltpu.VMEM_SHARED((NPAD, DIN), jnp.float32),
            pltpu.SemaphoreType.DMA,
            pltpu.SemaphoreType.DMA,
            pltpu.SemaphoreType.DMA,
            pltpu.SemaphoreType.DMA,
        ],
    )


def _make_deg():
    per_tile = (EPAD // 2) // NS
    n_chunks = per_tile // CHUNK

    def body(dst, ones, zinit, out, didx, onesbuf, acc, sem):
        cid = lax.axis_index("c")
        sid = lax.axis_index("s")

        def work(estart, core):
            cb = estart // CHUNK + sid * n_chunks
            pltpu.sync_copy(dst.at[pl.ds(cb, n_chunks)], didx)
            pltpu.sync_copy(ones, onesbuf)
            r0 = sid * ROWS_PER_TILE
            pltpu.sync_copy(zinit.at[pl.ds(r0, ROWS_PER_TILE)],
                            acc.at[pl.ds(r0, ROWS_PER_TILE)])
            plsc.subcore_barrier()

            @pl.loop(0, n_chunks)
            def _(j):
                pltpu.sync_copy(onesbuf, acc.at[didx.at[j]], add=True)

            plsc.subcore_barrier()
            pltpu.sync_copy(acc.at[pl.ds(r0, ROWS_PER_TILE)],
                            out.at[core, pl.ds(r0, ROWS_PER_TILE)])

        @pl.when(cid == 0)
        def _():
            work(0, 0)

        @pl.when(cid == 1)
        def _():
            work(EPAD // 2, 1)

    return pl.kernel(
        body,
        jax.ShapeDtypeStruct((2, NPAD, DIN), jnp.float32),
        mesh=_mesh(),
        scratch_types=[
            pltpu.VMEM((n_chunks, CHUNK), jnp.int32),
            pltpu.VMEM((CHUNK, DIN), jnp.float32),
            pltpu.VMEM_SHARED((NPAD, DIN), jnp.float32),
            pltpu.SemaphoreType.DMA,
        ],
    )


_agg_split = _make_agg(True)
_agg_full = _make_agg(False)
_deg = _make_deg()


def _blk(r, c):
    return pl.BlockSpec((r, c), lambda i: (i, 0))


def _full(r, c):
    return pl.BlockSpec((r, c), lambda i: (0, 0))


def _tc_a_body(x, w, b, dv, o):
    h = jax.nn.sigmoid(
        jnp.dot(x[...], w[...], preferred_element_type=jnp.float32) + b[...])
    o[...] = h * dv[...]


_tc_a = pl.pallas_call(
    _tc_a_body,
    out_shape=jax.ShapeDtypeStruct((NPAD, DIN), jnp.float32),
    grid=(NPAD // BR,),
    in_specs=[_blk(BR, DIN), _full(DIN, DIN), _full(1, DIN), _blk(BR, 1)],
    out_specs=_blk(BR, DIN),
)


def _tc_b_body(a0, a1, dv, w, b, o):
    t = (a0[...] + a1[...]) * dv[...]
    h = jnp.maximum(
        jnp.dot(t, w[...], preferred_element_type=jnp.float32) + b[...], 0.0)
    o[...] = h * dv[...]


_tc_b = pl.pallas_call(
    _tc_b_body,
    out_shape=jax.ShapeDtypeStruct((NPAD, DH), jnp.float32),
    grid=(NPAD // BR,),
    in_specs=[_blk(BR, DIN), _blk(BR, DIN), _blk(BR, 1),
              _full(DIN, DH), _full(1, DH)],
    out_specs=_blk(BR, DH),
)


def _tc_c_body(a0, a1, dv, w2a, w2b, b2, w3, o):
    t0 = a0[...] * dv[...]
    t1 = a1[...] * dv[...]
    h = jnp.maximum(
        jnp.dot(t0, w2a[...], preferred_element_type=jnp.float32)
        + jnp.dot(t1, w2b[...], preferred_element_type=jnp.float32)
        + b2[...], 0.0)
    o[...] = jnp.dot(h, w3[...], preferred_element_type=jnp.float32) * dv[...]


_tc_c = pl.pallas_call(
    _tc_c_body,
    out_shape=jax.ShapeDtypeStruct((NPAD, DOUT), jnp.float32),
    grid=(NPAD // BR,),
    in_specs=[_blk(BR, DIN), _blk(BR, DIN), _blk(BR, 1),
              _full(DIN, DH), _full(DIN, DH), _full(1, DH), _full(DH, DOUT)],
    out_specs=_blk(BR, DOUT),
)


def _tc_d_body(a0, a1, dv, b3, wo, bo, o):
    t = (a0[...] + a1[...]) * dv[...]
    h = jnp.maximum(t + b3[...], 0.0)
    o[...] = jnp.maximum(
        jnp.dot(h, wo[...], preferred_element_type=jnp.float32) + bo[...], 0.0)


_tc_d = pl.pallas_call(
    _tc_d_body,
    out_shape=jax.ShapeDtypeStruct((NPAD, DOUT), jnp.float32),
    grid=(NPAD // BR,),
    in_specs=[_blk(BR, DOUT), _blk(BR, DOUT), _blk(BR, 1),
              _full(1, DOUT), _full(DOUT, DOUT), _full(1, DOUT)],
    out_specs=_blk(BR, DOUT),
)


@jax.jit
def kernel(x, edge_index, W_in, b_in, W1, b1, W2, b2, W3, b3, W_out, b_out):
    f32 = jnp.float32
    src = edge_index[0]
    dst = edge_index[1]
    pad = N + (jnp.arange(EPAD - E0, dtype=jnp.int32) % (NPAD - N))
    src2d = jnp.concatenate([src, pad]).reshape(EPAD // CHUNK, CHUNK)
    dst2d = jnp.concatenate([dst, pad]).reshape(EPAD // CHUNK, CHUNK)
    xp = jnp.zeros((NPAD, DIN), f32).at[:N].set(x)
    ones128 = jnp.ones((CHUNK, DIN), f32)
    z128 = jnp.zeros((NPAD, DIN), f32)

    degp = _deg(dst2d, ones128, z128)
    deg = degp[0, :, 0] + degp[1, :, 0] + 1.0
    dv = lax.rsqrt(deg).reshape(NPAD, 1)

    g1 = _tc_a(xp, W_in, b_in.reshape(1, DIN), dv)
    a1 = _agg_split(g1, g1, src2d, dst2d, g1, z128)
    g2 = _tc_b(a1[0], a1[1], dv, W1, b1.reshape(1, DH))
    a2 = _agg_full(g2[:, :DIN], g2[:, DIN:], src2d, dst2d,
                   g2[:, :DIN], g2[:, DIN:])
    g3 = _tc_c(a2[0], a2[1], dv, W2[:DIN], W2[DIN:], b2.reshape(1, DH), W3)
    a3 = _agg_split(g3, g3, src2d, dst2d, g3, z128)
    y = _tc_d(a3[0], a3[1], dv, b3.reshape(1, DOUT), W_out,
              b_out.reshape(1, DOUT))
    return y[:N]


# speedup vs baseline: 3.0240x; 1.0007x over previous
import jax
import jax.numpy as jnp
from jax import lax
from jax.experimental import pallas as pl
from jax.experimental.pallas import tpu as pltpu
from jax.experimental.pallas import tpu_sc as plsc

N = 10000
DIN = 128
DH = 256
DOUT = 128
E0 = 320000

NC = 2
NS = 16
CHUNK = 128
ROWS_PER_TILE = 640
NPAD = NS * ROWS_PER_TILE
EPAD = 32768 * 10
BR = 2048


def _mesh():
    return plsc.VectorSubcoreMesh(core_axis_name="c", subcore_axis_name="s")


def _make_agg(split_edges: bool, idx_blk: int = 40):
    per_core = EPAD // 2 if split_edges else EPAD
    per_tile = per_core // NS
    n_chunks = per_tile // CHUNK
    n_outer = n_chunks // idx_blk
    estarts = (0, EPAD // 2) if split_edges else (0, 0)

    def body(t0, t1, src, dst, i0, i1, out, sidx, didx, bufa, bufb, acc,
             sema, semb, ssa, ssb):
        cid = lax.axis_index("c")
        sid = lax.axis_index("s")

        def work(table, init, estart, core):
            cb = estart // CHUNK + sid * n_chunks
            r0 = sid * ROWS_PER_TILE
            pltpu.sync_copy(init.at[pl.ds(r0, ROWS_PER_TILE)],
                            acc.at[pl.ds(r0, ROWS_PER_TILE)])
            plsc.subcore_barrier()

            for h in range(n_outer):
                pltpu.sync_copy(src.at[pl.ds(cb + h * idx_blk, idx_blk)], sidx)
                pltpu.sync_copy(dst.at[pl.ds(cb + h * idx_blk, idx_blk)], didx)
                pltpu.async_copy(table.at[sidx.at[0]], bufa, sema)

                @pl.loop(0, idx_blk, step=2)
                def _(j):
                    pltpu.make_async_copy(
                        table.at[sidx.at[j]], bufa, sema).wait()
                    pltpu.async_copy(bufa, acc.at[didx.at[j]], ssa, add=True)
                    pltpu.async_copy(table.at[sidx.at[j + 1]], bufb, semb)
                    pltpu.make_async_copy(
                        table.at[sidx.at[j + 1]], bufb, semb).wait()
                    pltpu.async_copy(bufb, acc.at[didx.at[j + 1]], ssb,
                                     add=True)
                    pltpu.make_async_copy(
                        bufa, acc.at[didx.at[j]], ssa).wait()

                    @pl.when(j + 2 < idx_blk)
                    def _():
                        pltpu.async_copy(table.at[sidx.at[j + 2]], bufa, sema)

                    pltpu.make_async_copy(
                        bufb, acc.at[didx.at[j + 1]], ssb).wait()

            plsc.subcore_barrier()
            pltpu.sync_copy(acc.at[pl.ds(r0, ROWS_PER_TILE)],
                            out.at[core, pl.ds(r0, ROWS_PER_TILE)])

        @pl.when(cid == 0)
        def _():
            work(t0, i0, estarts[0], 0)

        @pl.when(cid == 1)
        def _():
            work(t1, i1, estarts[1], 1)

    return pl.kernel(
        body,
        jax.ShapeDtypeStruct((2, NPAD, DIN), jnp.float32),
        mesh=_mesh(),
        scratch_types=[
            pltpu.VMEM((idx_blk, CHUNK), jnp.int32),
            pltpu.VMEM((idx_blk, CHUNK), jnp.int32),
            pltpu.VMEM((CHUNK, DIN), jnp.float32),
            pltpu.VMEM((CHUNK, DIN), jnp.float32),
            pltpu.VMEM_SHARED((NPAD, DIN), jnp.float32),
            pltpu.SemaphoreType.DMA,
            pltpu.SemaphoreType.DMA,
            pltpu.SemaphoreType.DMA,
            pltpu.SemaphoreType.DMA,
        ],
    )


def _make_deg(width: int = DIN):
    per_tile = (EPAD // 2) // NS
    n_chunks = per_tile // CHUNK

    def body(dst, ones, zinit, out, didx, onesbuf, acc, sem):
        cid = lax.axis_index("c")
        sid = lax.axis_index("s")

        def work(estart, core):
            cb = estart // CHUNK + sid * n_chunks
            pltpu.sync_copy(dst.at[pl.ds(cb, n_chunks)], didx)
            pltpu.sync_copy(ones, onesbuf)
            r0 = sid * ROWS_PER_TILE
            pltpu.sync_copy(zinit.at[pl.ds(r0, ROWS_PER_TILE)],
                            acc.at[pl.ds(r0, ROWS_PER_TILE)])
            plsc.subcore_barrier()

            @pl.loop(0, n_chunks)
            def _(j):
                pltpu.sync_copy(onesbuf, acc.at[didx.at[j]], add=True)

            plsc.subcore_barrier()
            pltpu.sync_copy(acc.at[pl.ds(r0, ROWS_PER_TILE)],
                            out.at[core, pl.ds(r0, ROWS_PER_TILE)])

        @pl.when(cid == 0)
        def _():
            work(0, 0)

        @pl.when(cid == 1)
        def _():
            work(EPAD // 2, 1)

    return pl.kernel(
        body,
        jax.ShapeDtypeStruct((2, NPAD, width), jnp.float32),
        mesh=_mesh(),
        scratch_types=[
            pltpu.VMEM((n_chunks, CHUNK), jnp.int32),
            pltpu.VMEM((CHUNK, width), jnp.float32),
            pltpu.VMEM_SHARED((NPAD, width), jnp.float32),
            pltpu.SemaphoreType.DMA,
        ],
    )


_agg_split = _make_agg(True)
_agg_full = _make_agg(False)
_deg = _make_deg()


def _blk(r, c):
    return pl.BlockSpec((r, c), lambda i: (i, 0))


def _full(r, c):
    return pl.BlockSpec((r, c), lambda i: (0, 0))


def _tc_a_body(x, w, b, dv, o):
    h = jax.nn.sigmoid(
        jnp.dot(x[...], w[...], preferred_element_type=jnp.float32) + b[...])
    o[...] = h * dv[...]


_tc_a = pl.pallas_call(
    _tc_a_body,
    out_shape=jax.ShapeDtypeStruct((NPAD, DIN), jnp.float32),
    grid=(NPAD // BR,),
    in_specs=[_blk(BR, DIN), _full(DIN, DIN), _full(1, DIN), _blk(BR, 1)],
    out_specs=_blk(BR, DIN),
)


def _tc_b_body(a0, a1, dv, w, b, o):
    t = (a0[...] + a1[...]) * dv[...]
    h = jnp.maximum(
        jnp.dot(t, w[...], preferred_element_type=jnp.float32) + b[...], 0.0)
    o[...] = h * dv[...]


_tc_b = pl.pallas_call(
    _tc_b_body,
    out_shape=jax.ShapeDtypeStruct((NPAD, DH), jnp.float32),
    grid=(NPAD // BR,),
    in_specs=[_blk(BR, DIN), _blk(BR, DIN), _blk(BR, 1),
              _full(DIN, DH), _full(1, DH)],
    out_specs=_blk(BR, DH),
)


def _tc_c_body(a0, a1, dv, w2a, w2b, b2, w3, o):
    t0 = a0[...] * dv[...]
    t1 = a1[...] * dv[...]
    h = jnp.maximum(
        jnp.dot(t0, w2a[...], preferred_element_type=jnp.float32)
        + jnp.dot(t1, w2b[...], preferred_element_type=jnp.float32)
        + b2[...], 0.0)
    o[...] = jnp.dot(h, w3[...], preferred_element_type=jnp.float32) * dv[...]


_tc_c = pl.pallas_call(
    _tc_c_body,
    out_shape=jax.ShapeDtypeStruct((NPAD, DOUT), jnp.float32),
    grid=(NPAD // BR,),
    in_specs=[_blk(BR, DIN), _blk(BR, DIN), _blk(BR, 1),
              _full(DIN, DH), _full(DIN, DH), _full(1, DH), _full(DH, DOUT)],
    out_specs=_blk(BR, DOUT),
)


def _tc_d_body(a0, a1, dv, b3, wo, bo, o):
    t = (a0[...] + a1[...]) * dv[...]
    h = jnp.maximum(t + b3[...], 0.0)
    o[...] = jnp.maximum(
        jnp.dot(h, wo[...], preferred_element_type=jnp.float32) + bo[...], 0.0)


_tc_d = pl.pallas_call(
    _tc_d_body,
    out_shape=jax.ShapeDtypeStruct((NPAD, DOUT), jnp.float32),
    grid=(NPAD // BR,),
    in_specs=[_blk(BR, DOUT), _blk(BR, DOUT), _blk(BR, 1),
              _full(1, DOUT), _full(DOUT, DOUT), _full(1, DOUT)],
    out_specs=_blk(BR, DOUT),
)


@jax.jit
def kernel(x, edge_index, W_in, b_in, W1, b1, W2, b2, W3, b3, W_out, b_out):
    f32 = jnp.float32
    src = edge_index[0]
    dst = edge_index[1]
    pad = N + (jnp.arange(EPAD - E0, dtype=jnp.int32) % (NPAD - N))
    src2d = jnp.concatenate([src, pad]).reshape(EPAD // CHUNK, CHUNK)
    dst2d = jnp.concatenate([dst, pad]).reshape(EPAD // CHUNK, CHUNK)
    xp = jnp.zeros((NPAD, DIN), f32).at[:N].set(x)
    ones128 = jnp.ones((CHUNK, DIN), f32)
    z128 = jnp.zeros((NPAD, DIN), f32)

    degp = _deg(dst2d, ones128, z128)
    deg = degp[0, :, 0] + degp[1, :, 0] + 1.0
    dv = lax.rsqrt(deg).reshape(NPAD, 1)

    g1 = _tc_a(xp, W_in, b_in.reshape(1, DIN), dv)
    a1 = _agg_split(g1, g1, src2d, dst2d, g1, z128)
    g2 = _tc_b(a1[0], a1[1], dv, W1, b1.reshape(1, DH))
    a2 = _agg_full(g2[:, :DIN], g2[:, DIN:], src2d, dst2d,
                   g2[:, :DIN], g2[:, DIN:])
    g3 = _tc_c(a2[0], a2[1], dv, W2[:DIN], W2[DIN:], b2.reshape(1, DH), W3)
    a3 = _agg_split(g3, g3, src2d, dst2d, g3, z128)
    y = _tc_d(a3[0], a3[1], dv, b3.reshape(1, DOUT), W_out,
              b_out.reshape(1, DOUT))
    return y[:N]
